# baseline (speedup 1.0000x reference)
import functools

import jax
import jax.numpy as jnp
from jax import lax
from jax.experimental import pallas as pl
from jax.experimental.pallas import tpu as pltpu
from jax.experimental.pallas import tpu_sc as plsc

B, S, D, E, I, K = 1, 2048, 1024, 8, 2048, 2
N = B * S
AUX_COEF = 0.01
T = 64
PAD_ROWS = N * K + E * T
NT = PAD_ROWS // T
NC, NS, L = 2, 16, 16
NW = NC * NS
CB = 256


def _gate_body(x_ref, gw_ref, pos_ref, p0_ref, p1_ref, v0_ref, v1_ref,
               eot_ref, aux_ref):
    x = x_ref[...]
    gw = gw_ref[...]
    st = lax.dot_general(gw, x, (((1,), (1,)), ((), ())))
    m = jnp.max(st, axis=0, keepdims=True)
    ex = jnp.exp(st - m)
    p = ex / jnp.sum(ex, axis=0, keepdims=True)
    eidx = lax.broadcasted_iota(jnp.int32, (E, N), 0)
    v0 = jnp.max(p, axis=0)
    i0 = jnp.min(jnp.where(p == v0[None, :], eidx, E), axis=0)
    pm = jnp.where(eidx == i0[None, :], -1.0, p)
    v1 = jnp.max(pm, axis=0)
    i1 = jnp.min(jnp.where(pm == v1[None, :], eidx, E), axis=0)
    oh0 = (eidx == i0[None, :]).astype(jnp.float32)
    oh1 = (eidx == i1[None, :]).astype(jnp.float32)

    tr = (lax.broadcasted_iota(jnp.int32, (CB, CB), 0) <
          lax.broadcasted_iota(jnp.int32, (CB, CB), 1)).astype(jnp.float32)
    off = jnp.zeros((E,), jnp.float32)
    r_parts = []
    for oh in (oh0, oh1):
        parts = []
        for b in range(N // CB):
            ohb = oh[:, b * CB:(b + 1) * CB]
            cb = lax.dot_general(ohb, tr, (((1,), (0,)), ((), ())),
                                 precision=lax.Precision.HIGHEST)
            cb = cb + off[:, None]
            parts.append(jnp.sum(ohb * cb, axis=0))
            off = off + jnp.sum(ohb, axis=1)
        r_parts.append(jnp.concatenate(parts, axis=0))
    cnt_tot = off
    ac = jnp.ceil(cnt_tot * (1.0 / T)) * float(T)
    stril = (lax.broadcasted_iota(jnp.int32, (E, E), 1) <
             lax.broadcasted_iota(jnp.int32, (E, E), 0)).astype(jnp.float32)
    po = lax.dot_general(stril, ac.reshape(E, 1), (((1,), (0,)), ((), ())),
                         precision=lax.Precision.HIGHEST)[:, 0]
    pos0 = jnp.sum(oh0 * po[:, None], axis=0) + r_parts[0]
    pos1 = jnp.sum(oh1 * po[:, None], axis=0) + r_parts[1]
    pos0i = pos0.astype(jnp.int32)
    pos1i = pos1.astype(jnp.int32)
    pos_ref[...] = jnp.concatenate(
        [pos0i.reshape(1, N), pos1i.reshape(1, N)], axis=0)
    p0_ref[...] = pos0i
    p1_ref[...] = pos1i
    v0_ref[...] = v0
    v1_ref[...] = v1
    tidx = lax.broadcasted_iota(jnp.int32, (E, NT), 1) * T
    eot_ref[...] = jnp.sum(
        (po[:, None].astype(jnp.int32) <= tidx).astype(jnp.int32), axis=0) - 1
    f = cnt_tot * (1.0 / (N * K))
    pmean = jnp.sum(p, axis=1) * (1.0 / N)
    aux_ref[0, 0] = (AUX_COEF * E) * jnp.sum(f * pmean)


def _gate_call(xf, gate_w):
    return pl.pallas_call(
        _gate_body,
        out_shape=(
            jax.ShapeDtypeStruct((2, N), jnp.int32),
            jax.ShapeDtypeStruct((N,), jnp.int32),
            jax.ShapeDtypeStruct((N,), jnp.int32),
            jax.ShapeDtypeStruct((N,), jnp.float32),
            jax.ShapeDtypeStruct((N,), jnp.float32),
            jax.ShapeDtypeStruct((NT,), jnp.int32),
            jax.ShapeDtypeStruct((1, 1), jnp.float32),
        ),
        in_specs=[pl.BlockSpec(memory_space=pltpu.VMEM),
                  pl.BlockSpec(memory_space=pltpu.VMEM)],
        out_specs=(pl.BlockSpec(memory_space=pltpu.VMEM),
                   pl.BlockSpec(memory_space=pltpu.VMEM),
                   pl.BlockSpec(memory_space=pltpu.VMEM),
                   pl.BlockSpec(memory_space=pltpu.VMEM),
                   pl.BlockSpec(memory_space=pltpu.VMEM),
                   pl.BlockSpec(memory_space=pltpu.VMEM),
                   pl.BlockSpec(memory_space=pltpu.SMEM)),
    )(xf, gate_w)


_SC_MESH = dict(core_axis_name="c", subcore_axis_name="s")
R_PER_W = PAD_ROWS // NW
GCH = 48
A_PER_TILE = (N * K) // NS
SCH = 128


def _dispatch_gather_body(posf_hbm, xf_hbm, xs_hbm,
                          vals_v, idx_v, dtok_sh, my_idx,
                          rows_a, rows_b, sem_a, sem_b):
    c = lax.axis_index("c")
    s = lax.axis_index("s")
    wid = c * NS + s
    for j in range(A_PER_TILE // SCH):
        flat0 = s * A_PER_TILE + j * SCH
        pltpu.sync_copy(posf_hbm.at[pl.ds(flat0, SCH)], idx_v)
        tok0 = lax.rem(flat0, N)
        for cc in range(SCH // L):
            vals_v[pl.ds(cc * L, L)] = lax.iota(jnp.int32, L) + (tok0 + cc * L)
        pltpu.sync_copy(vals_v, dtok_sh.at[idx_v])
    plsc.subcore_barrier()
    base = wid * R_PER_W
    pltpu.sync_copy(dtok_sh.at[pl.ds(base, R_PER_W)], my_idx)
    for q in range(R_PER_W // L):
        sl = pl.ds(q * L, L)
        v = my_idx[sl]
        my_idx[sl] = jnp.minimum(jnp.maximum(v, 0), N - 1)
    nch = R_PER_W // GCH
    bufs = (rows_a, rows_b)
    sems = (sem_a, sem_b)
    cp = pltpu.async_copy(xf_hbm.at[my_idx.at[pl.ds(0, GCH)]], rows_a, sem_a)
    for ci in range(nch):
        nxt = None
        if ci + 1 < nch:
            nxt = pltpu.async_copy(
                xf_hbm.at[my_idx.at[pl.ds((ci + 1) * GCH, GCH)]],
                bufs[(ci + 1) % 2], sems[(ci + 1) % 2])
        cp.wait()
        pltpu.sync_copy(bufs[ci % 2], xs_hbm.at[pl.ds(base + ci * GCH, GCH)])
        cp = nxt


def _dispatch_gather(posf, xf):
    return pl.kernel(
        _dispatch_gather_body,
        out_type=jax.ShapeDtypeStruct((PAD_ROWS, D), jnp.float32),
        mesh=plsc.VectorSubcoreMesh(**_SC_MESH),
        compiler_params=pltpu.CompilerParams(needs_layout_passes=False),
        scratch_types=[
            pltpu.VMEM((SCH,), jnp.int32),
            pltpu.VMEM((SCH,), jnp.int32),
            pltpu.VMEM_SHARED((PAD_ROWS,), jnp.int32),
            pltpu.VMEM((R_PER_W,), jnp.int32),
            pltpu.VMEM((GCH, D), jnp.float32),
            pltpu.VMEM((GCH, D), jnp.float32),
            pltpu.SemaphoreType.DMA,
            pltpu.SemaphoreType.DMA,
        ],
    )(posf, xf)


def _ffn_body(eot_ref, x_ref, w1_ref, b1_ref, w3_ref, b3_ref,
              w2_ref, b2_ref, o_ref):
    xb = x_ref[...]
    h1 = lax.dot_general(xb, w1_ref[0], (((1,), (1,)), ((), ()))) + b1_ref[0]
    h3 = lax.dot_general(xb, w3_ref[0], (((1,), (1,)), ((), ()))) + b3_ref[0]
    h = h1 * jax.nn.sigmoid(h1) * h3
    o = lax.dot_general(h, w2_ref[0], (((1,), (1,)), ((), ()))) + b2_ref[0]
    o_ref[...] = o


def _ffn_call(eot, xs, w1, b1, w3, b3, w2, b2):
    grid_spec = pltpu.PrefetchScalarGridSpec(
        num_scalar_prefetch=1,
        grid=(NT,),
        in_specs=[
            pl.BlockSpec((T, D), lambda t, eot: (t, 0)),
            pl.BlockSpec((1, I, D), lambda t, eot: (eot[t], 0, 0)),
            pl.BlockSpec((1, 1, I), lambda t, eot: (eot[t], 0, 0)),
            pl.BlockSpec((1, I, D), lambda t, eot: (eot[t], 0, 0)),
            pl.BlockSpec((1, 1, I), lambda t, eot: (eot[t], 0, 0)),
            pl.BlockSpec((1, D, I), lambda t, eot: (eot[t], 0, 0)),
            pl.BlockSpec((1, 1, D), lambda t, eot: (eot[t], 0, 0)),
        ],
        out_specs=pl.BlockSpec((T, D), lambda t, eot: (t, 0)),
    )
    return pl.pallas_call(
        _ffn_body,
        grid_spec=grid_spec,
        out_shape=jax.ShapeDtypeStruct((PAD_ROWS, D), jnp.float32),
    )(eot, xs, w1, b1.reshape(E, 1, I), w3, b3.reshape(E, 1, I),
      w2, b2.reshape(E, 1, D))


TOK_PER_W = N // NW
CCH = 16


def _combine_body(os_hbm, p0_hbm, p1_hbm, v0_hbm, v1_hbm, yf_hbm,
                  p0v, p1v, v0v, v1v, r0, r1, r0x, r1x, ob, obx,
                  sem0, sem1, sem_o):
    c = lax.axis_index("c")
    s = lax.axis_index("s")
    wid = c * NS + s
    base = wid * TOK_PER_W
    pltpu.sync_copy(p0_hbm.at[pl.ds(base, TOK_PER_W)], p0v)
    pltpu.sync_copy(p1_hbm.at[pl.ds(base, TOK_PER_W)], p1v)
    pltpu.sync_copy(v0_hbm.at[pl.ds(base, TOK_PER_W)], v0v)
    pltpu.sync_copy(v1_hbm.at[pl.ds(base, TOK_PER_W)], v1v)
    nch = TOK_PER_W // CCH
    r0b = (r0, r0x)
    r1b = (r1, r1x)
    obb = (ob, obx)
    sms = (sem0, sem1)
    cps = (pltpu.async_copy(os_hbm.at[p0v.at[pl.ds(0, CCH)]], r0, sem0),
           pltpu.async_copy(os_hbm.at[p1v.at[pl.ds(0, CCH)]], r1, sem0))
    out_cp = None
    for ci in range(nch):
        nxt = None
        if ci + 1 < nch:
            b = (ci + 1) % 2
            nxt = (pltpu.async_copy(
                       os_hbm.at[p0v.at[pl.ds((ci + 1) * CCH, CCH)]],
                       r0b[b], sms[b]),
                   pltpu.async_copy(
                       os_hbm.at[p1v.at[pl.ds((ci + 1) * CCH, CCH)]],
                       r1b[b], sms[b]))
        cps[0].wait()
        cps[1].wait()
        a = ci % 2
        ra, rb, oc = r0b[a], r1b[a], obb[a]
        vv0 = v0v[pl.ds(ci * CCH, CCH)]
        vv1 = v1v[pl.ds(ci * CCH, CCH)]
        for i in range(CCH):
            s0 = vv0[i]
            s1 = vv1[i]

            def fbody(j, carry, i=i, s0=s0, s1=s1, ra=ra, rb=rb, oc=oc):
                for u in range(4):
                    sl = pl.ds((j * 4 + u) * L, L)
                    oc[i, sl] = s0 * ra[i, sl] + s1 * rb[i, sl]
                return carry
            lax.fori_loop(0, D // (4 * L), fbody, 0)
        if out_cp is not None:
            out_cp.wait()
        out_cp = pltpu.async_copy(
            oc, yf_hbm.at[pl.ds(base + ci * CCH, CCH)], sem_o)
        cps = nxt
    out_cp.wait()


def _combine(os_rows, p0, p1, v0, v1):
    return pl.kernel(
        _combine_body,
        out_type=jax.ShapeDtypeStruct((N, D), jnp.float32),
        mesh=plsc.VectorSubcoreMesh(**_SC_MESH),
        compiler_params=pltpu.CompilerParams(needs_layout_passes=False),
        scratch_types=[
            pltpu.VMEM((TOK_PER_W,), jnp.int32),
            pltpu.VMEM((TOK_PER_W,), jnp.int32),
            pltpu.VMEM((TOK_PER_W,), jnp.float32),
            pltpu.VMEM((TOK_PER_W,), jnp.float32),
            pltpu.VMEM((CCH, D), jnp.float32),
            pltpu.VMEM((CCH, D), jnp.float32),
            pltpu.VMEM((CCH, D), jnp.float32),
            pltpu.VMEM((CCH, D), jnp.float32),
            pltpu.VMEM((CCH, D), jnp.float32),
            pltpu.VMEM((CCH, D), jnp.float32),
            pltpu.SemaphoreType.DMA,
            pltpu.SemaphoreType.DMA,
            pltpu.SemaphoreType.DMA,
        ],
    )(os_rows, p0, p1, v0, v1)


def kernel(x, gate_w, w1, b1, w2, b2, w3, b3):
    xf = x.reshape(N, D)
    pos2, p0, p1, v0, v1, eot, aux = _gate_call(xf, gate_w)
    xs = _dispatch_gather(pos2.reshape(N * K), xf)
    os_rows = _ffn_call(eot, xs, w1, b1, w3, b3, w2, b2)
    yf = _combine(os_rows, p0, p1, v0, v1)
    return yf.reshape(B, S, D), aux[0, 0]


# speedup vs baseline: 1.5159x; 1.5159x over previous
import functools

import jax
import jax.numpy as jnp
from jax import lax
from jax.experimental import pallas as pl
from jax.experimental.pallas import tpu as pltpu
from jax.experimental.pallas import tpu_sc as plsc

B, S, D, E, I, K = 1, 2048, 1024, 8, 2048, 2
N = B * S
AUX_COEF = 0.01
T = 256
PAD_ROWS = N * K + E * T
NT = PAD_ROWS // T
NC, NS, L = 2, 16, 16
NW = NC * NS
CB = 256


def _gate_body(x_ref, gw_ref, pos_ref, p0_ref, p1_ref, v0_ref, v1_ref,
               eot_ref, aux_ref):
    x = x_ref[...]
    gw = gw_ref[...]
    st = lax.dot_general(gw, x, (((1,), (1,)), ((), ())))
    m = jnp.max(st, axis=0, keepdims=True)
    ex = jnp.exp(st - m)
    p = ex / jnp.sum(ex, axis=0, keepdims=True)
    eidx = lax.broadcasted_iota(jnp.int32, (E, N), 0)
    v0 = jnp.max(p, axis=0)
    i0 = jnp.min(jnp.where(p == v0[None, :], eidx, E), axis=0)
    pm = jnp.where(eidx == i0[None, :], -1.0, p)
    v1 = jnp.max(pm, axis=0)
    i1 = jnp.min(jnp.where(pm == v1[None, :], eidx, E), axis=0)
    oh0 = (eidx == i0[None, :]).astype(jnp.float32)
    oh1 = (eidx == i1[None, :]).astype(jnp.float32)

    tr = (lax.broadcasted_iota(jnp.int32, (CB, CB), 0) <
          lax.broadcasted_iota(jnp.int32, (CB, CB), 1)).astype(jnp.float32)
    off = jnp.zeros((E,), jnp.float32)
    r_parts = []
    for oh in (oh0, oh1):
        parts = []
        for b in range(N // CB):
            ohb = oh[:, b * CB:(b + 1) * CB]
            cb = lax.dot_general(ohb, tr, (((1,), (0,)), ((), ())),
                                 precision=lax.Precision.HIGHEST)
            cb = cb + off[:, None]
            parts.append(jnp.sum(ohb * cb, axis=0))
            off = off + jnp.sum(ohb, axis=1)
        r_parts.append(jnp.concatenate(parts, axis=0))
    cnt_tot = off
    ac = jnp.ceil(cnt_tot * (1.0 / T)) * float(T)
    stril = (lax.broadcasted_iota(jnp.int32, (E, E), 1) <
             lax.broadcasted_iota(jnp.int32, (E, E), 0)).astype(jnp.float32)
    po = lax.dot_general(stril, ac.reshape(E, 1), (((1,), (0,)), ((), ())),
                         precision=lax.Precision.HIGHEST)[:, 0]
    pos0 = jnp.sum(oh0 * po[:, None], axis=0) + r_parts[0]
    pos1 = jnp.sum(oh1 * po[:, None], axis=0) + r_parts[1]
    pos0i = pos0.astype(jnp.int32)
    pos1i = pos1.astype(jnp.int32)
    pos_ref[...] = jnp.concatenate(
        [pos0i.reshape(1, N), pos1i.reshape(1, N)], axis=0)
    p0_ref[...] = pos0i
    p1_ref[...] = pos1i
    v0_ref[...] = v0
    v1_ref[...] = v1
    tidx = lax.broadcasted_iota(jnp.int32, (E, NT), 1) * T
    eot_ref[...] = jnp.sum(
        (po[:, None].astype(jnp.int32) <= tidx).astype(jnp.int32), axis=0) - 1
    f = cnt_tot * (1.0 / (N * K))
    pmean = jnp.sum(p, axis=1) * (1.0 / N)
    aux_ref[0, 0] = (AUX_COEF * E) * jnp.sum(f * pmean)


def _gate_call(xf, gate_w):
    return pl.pallas_call(
        _gate_body,
        out_shape=(
            jax.ShapeDtypeStruct((2, N), jnp.int32),
            jax.ShapeDtypeStruct((N,), jnp.int32),
            jax.ShapeDtypeStruct((N,), jnp.int32),
            jax.ShapeDtypeStruct((N,), jnp.float32),
            jax.ShapeDtypeStruct((N,), jnp.float32),
            jax.ShapeDtypeStruct((NT,), jnp.int32),
            jax.ShapeDtypeStruct((1, 1), jnp.float32),
        ),
        in_specs=[pl.BlockSpec(memory_space=pltpu.VMEM),
                  pl.BlockSpec(memory_space=pltpu.VMEM)],
        out_specs=(pl.BlockSpec(memory_space=pltpu.VMEM),
                   pl.BlockSpec(memory_space=pltpu.VMEM),
                   pl.BlockSpec(memory_space=pltpu.VMEM),
                   pl.BlockSpec(memory_space=pltpu.VMEM),
                   pl.BlockSpec(memory_space=pltpu.VMEM),
                   pl.BlockSpec(memory_space=pltpu.VMEM),
                   pl.BlockSpec(memory_space=pltpu.SMEM)),
    )(xf, gate_w)


_SC_MESH = dict(core_axis_name="c", subcore_axis_name="s")
R_PER_W = PAD_ROWS // NW
GCH = 48
A_PER_TILE = (N * K) // NS
SCH = 128


def _dispatch_gather_body(posf_hbm, xf_hbm, xs_hbm,
                          vals_v, idx_v, dtok_sh, my_idx,
                          rows_a, rows_b, sem_a, sem_b):
    c = lax.axis_index("c")
    s = lax.axis_index("s")
    wid = c * NS + s
    for j in range(A_PER_TILE // SCH):
        flat0 = s * A_PER_TILE + j * SCH
        pltpu.sync_copy(posf_hbm.at[pl.ds(flat0, SCH)], idx_v)
        tok0 = lax.rem(flat0, N)
        for cc in range(SCH // L):
            vals_v[pl.ds(cc * L, L)] = lax.iota(jnp.int32, L) + (tok0 + cc * L)
        pltpu.sync_copy(vals_v, dtok_sh.at[idx_v])
    plsc.subcore_barrier()
    base = wid * R_PER_W
    pltpu.sync_copy(dtok_sh.at[pl.ds(base, R_PER_W)], my_idx)
    for q in range(R_PER_W // L):
        sl = pl.ds(q * L, L)
        v = my_idx[sl]
        my_idx[sl] = jnp.minimum(jnp.maximum(v, 0), N - 1)
    nch = R_PER_W // GCH
    bufs = (rows_a, rows_b)
    sems = (sem_a, sem_b)
    cp = pltpu.async_copy(xf_hbm.at[my_idx.at[pl.ds(0, GCH)]], rows_a, sem_a)
    for ci in range(nch):
        nxt = None
        if ci + 1 < nch:
            nxt = pltpu.async_copy(
                xf_hbm.at[my_idx.at[pl.ds((ci + 1) * GCH, GCH)]],
                bufs[(ci + 1) % 2], sems[(ci + 1) % 2])
        cp.wait()
        pltpu.sync_copy(bufs[ci % 2], xs_hbm.at[pl.ds(base + ci * GCH, GCH)])
        cp = nxt


def _dispatch_gather(posf, xf):
    return pl.kernel(
        _dispatch_gather_body,
        out_type=jax.ShapeDtypeStruct((PAD_ROWS, D), jnp.float32),
        mesh=plsc.VectorSubcoreMesh(**_SC_MESH),
        compiler_params=pltpu.CompilerParams(needs_layout_passes=False),
        scratch_types=[
            pltpu.VMEM((SCH,), jnp.int32),
            pltpu.VMEM((SCH,), jnp.int32),
            pltpu.VMEM_SHARED((PAD_ROWS,), jnp.int32),
            pltpu.VMEM((R_PER_W,), jnp.int32),
            pltpu.VMEM((GCH, D), jnp.float32),
            pltpu.VMEM((GCH, D), jnp.float32),
            pltpu.SemaphoreType.DMA,
            pltpu.SemaphoreType.DMA,
        ],
    )(posf, xf)


def _ffn_body(eot_ref, x_ref, w1_ref, b1_ref, w3_ref, b3_ref,
              w2_ref, b2_ref, o_ref):
    xb = x_ref[...]
    h1 = lax.dot_general(xb, w1_ref[0], (((1,), (1,)), ((), ()))) + b1_ref[0]
    h3 = lax.dot_general(xb, w3_ref[0], (((1,), (1,)), ((), ()))) + b3_ref[0]
    h = h1 * jax.nn.sigmoid(h1) * h3
    o = lax.dot_general(h, w2_ref[0], (((1,), (1,)), ((), ()))) + b2_ref[0]
    o_ref[...] = o


def _ffn_call(eot, xs, w1, b1, w3, b3, w2, b2):
    grid_spec = pltpu.PrefetchScalarGridSpec(
        num_scalar_prefetch=1,
        grid=(NT,),
        in_specs=[
            pl.BlockSpec((T, D), lambda t, eot: (t, 0)),
            pl.BlockSpec((1, I, D), lambda t, eot: (eot[t], 0, 0)),
            pl.BlockSpec((1, 1, I), lambda t, eot: (eot[t], 0, 0)),
            pl.BlockSpec((1, I, D), lambda t, eot: (eot[t], 0, 0)),
            pl.BlockSpec((1, 1, I), lambda t, eot: (eot[t], 0, 0)),
            pl.BlockSpec((1, D, I), lambda t, eot: (eot[t], 0, 0)),
            pl.BlockSpec((1, 1, D), lambda t, eot: (eot[t], 0, 0)),
        ],
        out_specs=pl.BlockSpec((T, D), lambda t, eot: (t, 0)),
    )
    return pl.pallas_call(
        _ffn_body,
        grid_spec=grid_spec,
        out_shape=jax.ShapeDtypeStruct((PAD_ROWS, D), jnp.float32),
    )(eot, xs, w1, b1.reshape(E, 1, I), w3, b3.reshape(E, 1, I),
      w2, b2.reshape(E, 1, D))


TOK_PER_W = N // NW
CCH = 16


def _combine_body(os_hbm, p0_hbm, p1_hbm, v0_hbm, v1_hbm, yf_hbm,
                  p0v, p1v, v0v, v1v, r0, r1, r0x, r1x, ob, obx,
                  sem0, sem1, sem_o):
    c = lax.axis_index("c")
    s = lax.axis_index("s")
    wid = c * NS + s
    base = wid * TOK_PER_W
    pltpu.sync_copy(p0_hbm.at[pl.ds(base, TOK_PER_W)], p0v)
    pltpu.sync_copy(p1_hbm.at[pl.ds(base, TOK_PER_W)], p1v)
    pltpu.sync_copy(v0_hbm.at[pl.ds(base, TOK_PER_W)], v0v)
    pltpu.sync_copy(v1_hbm.at[pl.ds(base, TOK_PER_W)], v1v)
    nch = TOK_PER_W // CCH
    r0b = (r0, r0x)
    r1b = (r1, r1x)
    obb = (ob, obx)
    sms = (sem0, sem1)
    cps = (pltpu.async_copy(os_hbm.at[p0v.at[pl.ds(0, CCH)]], r0, sem0),
           pltpu.async_copy(os_hbm.at[p1v.at[pl.ds(0, CCH)]], r1, sem0))
    out_cp = None
    for ci in range(nch):
        nxt = None
        if ci + 1 < nch:
            b = (ci + 1) % 2
            nxt = (pltpu.async_copy(
                       os_hbm.at[p0v.at[pl.ds((ci + 1) * CCH, CCH)]],
                       r0b[b], sms[b]),
                   pltpu.async_copy(
                       os_hbm.at[p1v.at[pl.ds((ci + 1) * CCH, CCH)]],
                       r1b[b], sms[b]))
        cps[0].wait()
        cps[1].wait()
        a = ci % 2
        ra, rb, oc = r0b[a], r1b[a], obb[a]
        vv0 = v0v[pl.ds(ci * CCH, CCH)]
        vv1 = v1v[pl.ds(ci * CCH, CCH)]
        for i in range(CCH):
            s0 = vv0[i]
            s1 = vv1[i]

            def fbody(j, carry, i=i, s0=s0, s1=s1, ra=ra, rb=rb, oc=oc):
                for u in range(4):
                    sl = pl.ds((j * 4 + u) * L, L)
                    oc[i, sl] = s0 * ra[i, sl] + s1 * rb[i, sl]
                return carry
            lax.fori_loop(0, D // (4 * L), fbody, 0)
        if out_cp is not None:
            out_cp.wait()
        out_cp = pltpu.async_copy(
            oc, yf_hbm.at[pl.ds(base + ci * CCH, CCH)], sem_o)
        cps = nxt
    out_cp.wait()


def _combine(os_rows, p0, p1, v0, v1):
    return pl.kernel(
        _combine_body,
        out_type=jax.ShapeDtypeStruct((N, D), jnp.float32),
        mesh=plsc.VectorSubcoreMesh(**_SC_MESH),
        compiler_params=pltpu.CompilerParams(needs_layout_passes=False),
        scratch_types=[
            pltpu.VMEM((TOK_PER_W,), jnp.int32),
            pltpu.VMEM((TOK_PER_W,), jnp.int32),
            pltpu.VMEM((TOK_PER_W,), jnp.float32),
            pltpu.VMEM((TOK_PER_W,), jnp.float32),
            pltpu.VMEM((CCH, D), jnp.float32),
            pltpu.VMEM((CCH, D), jnp.float32),
            pltpu.VMEM((CCH, D), jnp.float32),
            pltpu.VMEM((CCH, D), jnp.float32),
            pltpu.VMEM((CCH, D), jnp.float32),
            pltpu.VMEM((CCH, D), jnp.float32),
            pltpu.SemaphoreType.DMA,
            pltpu.SemaphoreType.DMA,
            pltpu.SemaphoreType.DMA,
        ],
    )(os_rows, p0, p1, v0, v1)


def kernel(x, gate_w, w1, b1, w2, b2, w3, b3):
    xf = x.reshape(N, D)
    pos2, p0, p1, v0, v1, eot, aux = _gate_call(xf, gate_w)
    xs = _dispatch_gather(pos2.reshape(N * K), xf)
    os_rows = _ffn_call(eot, xs, w1, b1, w3, b3, w2, b2)
    yf = _combine(os_rows, p0, p1, v0, v1)
    return yf.reshape(B, S, D), aux[0, 0]
